# baseline (speedup 1.0000x reference)
import jax
import jax.numpy as jnp
from jax.experimental import pallas as pl
from jax.experimental.pallas import tpu as pltpu

_B, _S, _D = 16, 1024, 256
_ROWS = _B * _S
_BLK = 16384


def _copy_kernel(x_ref, o_ref):
    o_ref[...] = x_ref[...]


def kernel(x, W):
    del W
    flat = x.reshape(_ROWS, _D)
    out = pl.pallas_call(
        _copy_kernel,
        grid=(_ROWS // _BLK,),
        in_specs=[pl.BlockSpec((_BLK, _D), lambda i: (i, 0))],
        out_specs=pl.BlockSpec((_BLK, _D), lambda i: (i, 0)),
        out_shape=jax.ShapeDtypeStruct((_ROWS, _D), x.dtype),
        compiler_params=pltpu.CompilerParams(
            dimension_semantics=("parallel",),
        ),
    )(flat)
    return out.reshape(x.shape)


# speedup vs baseline: 1.1989x; 1.1989x over previous
import jax
import jax.numpy as jnp
from jax.experimental import pallas as pl
from jax.experimental.pallas import tpu as pltpu

_B, _S, _D = 16, 1024, 256
_ROWS = _B * _S
_K = 4
_CH = _ROWS // _K


def _copy_kernel(x_hbm, o_hbm, buf, insems, outsems):
    for k in range(_K):
        pltpu.make_async_copy(
            x_hbm.at[pl.ds(k * _CH, _CH), :], buf.at[k], insems.at[k]
        ).start()
    for k in range(_K):
        pltpu.make_async_copy(
            x_hbm.at[pl.ds(k * _CH, _CH), :], buf.at[k], insems.at[k]
        ).wait()
        pltpu.make_async_copy(
            buf.at[k], o_hbm.at[pl.ds(k * _CH, _CH), :], outsems.at[k]
        ).start()
    for k in range(_K):
        pltpu.make_async_copy(
            buf.at[k], o_hbm.at[pl.ds(k * _CH, _CH), :], outsems.at[k]
        ).wait()


def kernel(x, W):
    del W
    flat = x.reshape(_ROWS, _D)
    out = pl.pallas_call(
        _copy_kernel,
        in_specs=[pl.BlockSpec(memory_space=pltpu.MemorySpace.HBM)],
        out_specs=pl.BlockSpec(memory_space=pltpu.MemorySpace.HBM),
        out_shape=jax.ShapeDtypeStruct((_ROWS, _D), x.dtype),
        scratch_shapes=[
            pltpu.VMEM((_K, _CH, _D), x.dtype),
            pltpu.SemaphoreType.DMA((_K,)),
            pltpu.SemaphoreType.DMA((_K,)),
        ],
    )(flat)
    return out.reshape(x.shape)
